# Initial kernel scaffold; baseline (speedup 1.0000x reference)
import jax
import jax.numpy as jnp
from jax.experimental import pallas as pl


def kernel(x, idx, dis, angle_t_sin, angle_t_cos):
    raise NotImplementedError("write your pallas kernel here")


# speedup vs baseline: 2.5245x; 2.5245x over previous
import functools

import jax
import jax.numpy as jnp
from jax import lax
from jax.experimental import pallas as pl
from jax.experimental.pallas import tpu as pltpu
from jax.experimental.pallas import tpu_sc as plsc

D = 128
K = 16
Q = 16384
GW = K * D
OUT_W = GW + 3 * K
NC, NS = 2, 16
NW = NC * NS
QPW = Q // NW
C = 8
NCH = QPW // C


def _build_kernel():
  mesh = plsc.VectorSubcoreMesh(
      core_axis_name="c", subcore_axis_name="s", num_cores=NC, num_subcores=NS
  )

  @functools.partial(
      pl.kernel,
      out_type=jax.ShapeDtypeStruct((Q, OUT_W), jnp.float32),
      mesh=mesh,
      scratch_types=[
          pltpu.VMEM((C * K,), jnp.int32),
          pltpu.VMEM((C * K, D), jnp.float32),
          pltpu.VMEM((C * K,), jnp.float32),
          pltpu.VMEM((C * K,), jnp.float32),
          pltpu.VMEM((C * K,), jnp.float32),
          pltpu.VMEM((C, OUT_W), jnp.float32),
          pltpu.SemaphoreType.DMA,
      ],
  )
  def run(x_hbm, idx_hbm, dis_hbm, sin_hbm, cos_hbm, out_hbm, idx_v,
          rows_v, fd_v, fs_v, fc_v, fcat_v, sem):
    wid = lax.axis_index("s") * NC + lax.axis_index("c")
    qbase = wid * QPW

    @pl.loop(0, NCH)
    def _(i):
      q0 = qbase + i * C
      pltpu.sync_copy(idx_hbm.at[pl.ds(q0 * K, C * K)], idx_v)
      pltpu.async_copy(x_hbm.at[idx_v], rows_v, sem).wait()
      pltpu.sync_copy(
          rows_v.reshape(C, GW), out_hbm.at[pl.ds(q0, C), pl.ds(0, GW)]
      )
      pltpu.sync_copy(dis_hbm.at[pl.ds(q0 * K, C * K)], fd_v)
      pltpu.sync_copy(sin_hbm.at[pl.ds(q0 * K, C * K)], fs_v)
      pltpu.sync_copy(cos_hbm.at[pl.ds(q0 * K, C * K)], fc_v)
      for q in range(C):
        fcat_v[q, pl.ds(GW, K)] = fd_v[pl.ds(q * K, K)]
        fcat_v[q, pl.ds(GW + K, K)] = fs_v[pl.ds(q * K, K)]
        fcat_v[q, pl.ds(GW + 2 * K, K)] = fc_v[pl.ds(q * K, K)]
      pltpu.sync_copy(
          fcat_v.at[:, pl.ds(GW, 3 * K)],
          out_hbm.at[pl.ds(q0, C), pl.ds(GW, 3 * K)],
      )

  return run


def kernel(x, idx, dis, angle_t_sin, angle_t_cos):
  idx_flat = idx.reshape(-1).astype(jnp.int32)
  run = _build_kernel()
  return run(
      x,
      idx_flat,
      dis.reshape(-1),
      angle_t_sin.reshape(-1),
      angle_t_cos.reshape(-1),
  )


# speedup vs baseline: 3.7642x; 1.4911x over previous
import functools

import jax
import jax.numpy as jnp
from jax import lax
from jax.experimental import pallas as pl
from jax.experimental.pallas import tpu as pltpu
from jax.experimental.pallas import tpu_sc as plsc

D = 128
K = 16
Q = 16384
GW = K * D
OUT_W = GW + 3 * K
NC, NS = 2, 16
NW = NC * NS
QPW = Q // NW
C = 8
NCH = QPW // C
NBUF = 2


def _build_kernel():
  mesh = plsc.VectorSubcoreMesh(
      core_axis_name="c", subcore_axis_name="s", num_cores=NC, num_subcores=NS
  )

  @functools.partial(
      pl.kernel,
      out_type=jax.ShapeDtypeStruct((Q, OUT_W), jnp.float32),
      mesh=mesh,
      scratch_types=[
          pltpu.VMEM((QPW * K,), jnp.int32),
          pltpu.VMEM((QPW * K,), jnp.float32),
          pltpu.VMEM((QPW * K,), jnp.float32),
          pltpu.VMEM((QPW * K,), jnp.float32),
          [pltpu.VMEM((C * K, D), jnp.float32) for _ in range(NBUF)],
          [pltpu.VMEM((C, OUT_W), jnp.float32) for _ in range(NBUF)],
          [pltpu.SemaphoreType.DMA for _ in range(NBUF)],
          [pltpu.SemaphoreType.DMA for _ in range(NBUF)],
          [pltpu.SemaphoreType.DMA for _ in range(NBUF)],
      ],
  )
  def run(x_hbm, idx_hbm, dis_hbm, sin_hbm, cos_hbm, out_hbm, idx_all,
          fd_all, fs_all, fc_all, rows_v, fcat_v, sem_g, sem_w, sem_t):
    wid = lax.axis_index("s") * NC + lax.axis_index("c")
    qbase = wid * QPW

    pltpu.sync_copy(idx_hbm.at[pl.ds(qbase * K, QPW * K)], idx_all)
    pltpu.sync_copy(dis_hbm.at[pl.ds(qbase * K, QPW * K)], fd_all)
    pltpu.sync_copy(sin_hbm.at[pl.ds(qbase * K, QPW * K)], fs_all)
    pltpu.sync_copy(cos_hbm.at[pl.ds(qbase * K, QPW * K)], fc_all)

    def chunk_refs(i, b):
      q0 = qbase + i * C
      rows_src = rows_v[b].reshape(C, GW)
      rows_dst = out_hbm.at[pl.ds(q0, C), pl.ds(0, GW)]
      tail_src = fcat_v[b].at[:, pl.ds(GW, 3 * K)]
      tail_dst = out_hbm.at[pl.ds(q0, C), pl.ds(GW, 3 * K)]
      return rows_src, rows_dst, tail_src, tail_dst

    @pl.loop(0, NCH // NBUF)
    def _(g):
      for b in range(NBUF):
        i = g * NBUF + b
        rows_src, rows_dst, tail_src, tail_dst = chunk_refs(i, b)

        @pl.when(g > 0)
        def _():
          pltpu.make_async_copy(rows_src, rows_dst, sem_w[b]).wait()
          pltpu.make_async_copy(tail_src, tail_dst, sem_t[b]).wait()

        gcp = pltpu.async_copy(
            x_hbm.at[idx_all.at[pl.ds(i * C * K, C * K)]], rows_v[b], sem_g[b]
        )
        for q in range(C):
          off = (i * C + q) * K
          fcat_v[b][q, pl.ds(GW, K)] = fd_all[pl.ds(off, K)]
          fcat_v[b][q, pl.ds(GW + K, K)] = fs_all[pl.ds(off, K)]
          fcat_v[b][q, pl.ds(GW + 2 * K, K)] = fc_all[pl.ds(off, K)]
        gcp.wait()

        pltpu.async_copy(rows_src, rows_dst, sem_w[b])
        pltpu.async_copy(tail_src, tail_dst, sem_t[b])

    for b in range(NBUF):
      i = NCH - NBUF + b
      rows_src, rows_dst, tail_src, tail_dst = chunk_refs(i, b)
      pltpu.make_async_copy(rows_src, rows_dst, sem_w[b]).wait()
      pltpu.make_async_copy(tail_src, tail_dst, sem_t[b]).wait()

  return run


def kernel(x, idx, dis, angle_t_sin, angle_t_cos):
  idx_flat = idx.reshape(-1).astype(jnp.int32)
  run = _build_kernel()
  return run(
      x,
      idx_flat,
      dis.reshape(-1),
      angle_t_sin.reshape(-1),
      angle_t_cos.reshape(-1),
  )


# speedup vs baseline: 3.8078x; 1.0116x over previous
import functools

import jax
import jax.numpy as jnp
from jax import lax
from jax.experimental import pallas as pl
from jax.experimental.pallas import tpu as pltpu
from jax.experimental.pallas import tpu_sc as plsc

D = 128
K = 16
Q = 16384
GW = K * D
OUT_W = GW + 3 * K
NC, NS = 2, 16
NW = NC * NS
QPW = Q // NW
C = 8
NCH = QPW // C
NBUF = 2


def _build_kernel():
  mesh = plsc.VectorSubcoreMesh(
      core_axis_name="c", subcore_axis_name="s", num_cores=NC, num_subcores=NS
  )

  @functools.partial(
      pl.kernel,
      out_type=jax.ShapeDtypeStruct((Q, OUT_W), jnp.float32),
      mesh=mesh,
      scratch_types=[
          pltpu.VMEM((QPW * K // 128, 128), jnp.float32),
          pltpu.VMEM((QPW * K // 128, 128), jnp.float32),
          pltpu.VMEM((QPW * K // 128, 128), jnp.float32),
          pltpu.VMEM((QPW * K // 128, 128), jnp.float32),
          [pltpu.VMEM((C * K, D), jnp.float32) for _ in range(NBUF)],
          [pltpu.VMEM((C, OUT_W), jnp.float32) for _ in range(NBUF)],
          [pltpu.SemaphoreType.DMA for _ in range(NBUF)],
          [pltpu.SemaphoreType.DMA for _ in range(NBUF)],
          [pltpu.SemaphoreType.DMA for _ in range(NBUF)],
      ],
  )
  def run(x_hbm, cat_hbm, out_hbm, idx_all,
          fd_all, fs_all, fc_all, rows_v, fcat_v, sem_g, sem_w, sem_t):
    wid = lax.axis_index("s") * NC + lax.axis_index("c")
    qbase = wid * QPW

    R = QPW * K // 128
    QR = Q * K // 128
    rbase = wid * R
    pltpu.sync_copy(cat_hbm.at[pl.ds(rbase, R), :], idx_all)
    pltpu.sync_copy(cat_hbm.at[pl.ds(QR + rbase, R), :], fd_all)
    pltpu.sync_copy(cat_hbm.at[pl.ds(2 * QR + rbase, R), :], fs_all)
    pltpu.sync_copy(cat_hbm.at[pl.ds(3 * QR + rbase, R), :], fc_all)

    def chunk_refs(i, b):
      q0 = qbase + i * C
      rows_src = rows_v[b].reshape(C, GW)
      rows_dst = out_hbm.at[pl.ds(q0, C), pl.ds(0, GW)]
      tail_src = fcat_v[b].at[:, pl.ds(GW, 3 * K)]
      tail_dst = out_hbm.at[pl.ds(q0, C), pl.ds(GW, 3 * K)]
      return rows_src, rows_dst, tail_src, tail_dst

    @pl.loop(0, NCH // NBUF)
    def _(g):
      for b in range(NBUF):
        i = g * NBUF + b
        rows_src, rows_dst, tail_src, tail_dst = chunk_refs(i, b)

        @pl.when(g > 0)
        def _():
          pltpu.make_async_copy(rows_src, rows_dst, sem_w[b]).wait()
          pltpu.make_async_copy(tail_src, tail_dst, sem_t[b]).wait()

        gcp = pltpu.async_copy(
            x_hbm.at[idx_all.bitcast(jnp.int32).at[i]],
            rows_v[b],
            sem_g[b],
        )
        for q in range(C):
          fcat_v[b][q, pl.ds(GW, K)] = fd_all[i, pl.ds(q * K, K)]
          fcat_v[b][q, pl.ds(GW + K, K)] = fs_all[i, pl.ds(q * K, K)]
          fcat_v[b][q, pl.ds(GW + 2 * K, K)] = fc_all[i, pl.ds(q * K, K)]
        gcp.wait()

        pltpu.async_copy(rows_src, rows_dst, sem_w[b])
        pltpu.async_copy(tail_src, tail_dst, sem_t[b])

    for b in range(NBUF):
      i = NCH - NBUF + b
      rows_src, rows_dst, tail_src, tail_dst = chunk_refs(i, b)
      pltpu.make_async_copy(rows_src, rows_dst, sem_w[b]).wait()
      pltpu.make_async_copy(tail_src, tail_dst, sem_t[b]).wait()

  return run


def kernel(x, idx, dis, angle_t_sin, angle_t_cos):
  idx_f = jax.lax.bitcast_convert_type(idx.astype(jnp.int32), jnp.float32)
  cat = jnp.concatenate(
      [idx_f, dis, angle_t_sin, angle_t_cos], axis=0
  ).reshape(4 * Q * K // 128, 128)
  run = _build_kernel()
  return run(x, cat)
